# Initial kernel scaffold; baseline (speedup 1.0000x reference)
import jax
import jax.numpy as jnp
from jax.experimental import pallas as pl


def kernel(x_transaction, x_wallet, edge_index_tt, edge_index_ww, edge_index_wt, edge_index_tw, tx_lin_W, tx_lin_b, wallet_lin_W, wallet_lin_b, Wl_tt, bl_tt, Wr_tt, Wl_ww, bl_ww, Wr_ww, Wl_wt, bl_wt, Wr_wt, Wl_tw, bl_tw, Wr_tw, tx_c1_W, tx_c1_b, tx_c2_W, tx_c2_b, w_c1_W, w_c1_b, w_c2_W, w_c2_b):
    raise NotImplementedError("write your pallas kernel here")


# speedup vs baseline: 1.6848x; 1.6848x over previous
import functools

import jax
import jax.numpy as jnp
from jax import lax
from jax.experimental import pallas as pl
from jax.experimental.pallas import tpu as pltpu
from jax.experimental.pallas import tpu_sc as plsc

N = 50000
D = 128
E = 500000
HC = 64
NC_CLS = 2

NCORE = 2
NSUB = 16
LANES = 16
NSH = 4
DS = D // NSH

EPAD = 503808
CH_AGG = EPAD // NSUB
CH_CNT = EPAD // (2 * NSUB)
G = 128
NP = 50048
ACC_R = 50176
TRASH = NP
WBR = NP // NSUB
ZR = ACC_R // NSUB
ZB = 784

_mesh = plsc.VectorSubcoreMesh(core_axis_name="c", subcore_axis_name="s")


@functools.partial(
    pl.kernel,
    out_type=jax.ShapeDtypeStruct((2, NP, LANES), jnp.float32),
    mesh=_mesh,
    scratch_types=[
        pltpu.VMEM((G,), jnp.int32),
        pltpu.VMEM((G, LANES), jnp.float32),
        pltpu.VMEM((ZB, LANES), jnp.float32),
        pltpu.VMEM_SHARED((ACC_R, LANES), jnp.float32),
    ],
    compiler_params=pltpu.CompilerParams(use_tc_tiling_on_sc=False),
)
def _cnt_sc(dst_hbm, out_hbm, didx, obuf, zbuf, cacc):
    cid = lax.axis_index("c")
    sid = lax.axis_index("s")
    zero = jnp.zeros((LANES,), jnp.float32)
    one = jnp.ones((LANES,), jnp.float32)

    def prow(r, c):
        obuf[r, pl.ds(0, LANES)] = one
        return c

    lax.fori_loop(0, G, prow, 0)

    def zrow(r, c):
        zbuf[r, pl.ds(0, LANES)] = zero
        return c

    lax.fori_loop(0, ZB, zrow, 0)

    for k in range(ZR // ZB):
        pltpu.sync_copy(zbuf, cacc.at[pl.ds(sid * ZR + k * ZB, ZB)])
    plsc.subcore_barrier()

    base = (cid * NSUB + sid) * CH_CNT

    def body(j, c):
        pltpu.sync_copy(dst_hbm.at[pl.ds(base + j * G, G)], didx)
        pltpu.sync_copy(obuf, cacc.at[didx], add=True)
        return c

    lax.fori_loop(0, CH_CNT // G, body, 0)
    plsc.subcore_barrier()
    pltpu.sync_copy(cacc.at[pl.ds(sid * WBR, WBR)],
                    out_hbm.at[cid, pl.ds(sid * WBR, WBR)])


@functools.partial(
    pl.kernel,
    out_type=[jax.ShapeDtypeStruct((NP, DS), jnp.float32)
              for _ in range(NSH)],
    mesh=_mesh,
    scratch_types=[
        pltpu.VMEM((G,), jnp.int32),
        pltpu.VMEM((G,), jnp.int32),
        pltpu.VMEM((G, DS), jnp.float32),
        pltpu.VMEM((ZB, DS), jnp.float32),
        pltpu.VMEM_SHARED((ACC_R, DS), jnp.float32),
        pltpu.SemaphoreType.DMA,
    ],
    compiler_params=pltpu.CompilerParams(use_tc_tiling_on_sc=False),
)
def _agg_sc(h0, h1, h2, h3, src_hbm, dst_hbm, o0, o1, o2, o3,
            sidx, didx, gbuf, zbuf, acc, sem):
    cid = lax.axis_index("c")
    sid = lax.axis_index("s")
    base = sid * CH_AGG
    zero = jnp.zeros((LANES,), jnp.float32)

    def zrow(r, c):
        for t in range(DS // LANES):
            zbuf[r, pl.ds(t * LANES, LANES)] = zero
        return c

    lax.fori_loop(0, ZB, zrow, 0)

    hs = (h0, h1, h2, h3)
    os = (o0, o1, o2, o3)
    for k in range(NSH):
        @pl.when(cid == k // 2)
        def _(k=k):
            h_hbm = hs[k]
            out_hbm = os[k]
            for t in range(ZR // ZB):
                pltpu.sync_copy(zbuf, acc.at[pl.ds(sid * ZR + t * ZB, ZB)])
            plsc.subcore_barrier()

            def body(j, c):
                pltpu.sync_copy(src_hbm.at[pl.ds(base + j * G, G)], sidx)
                pltpu.sync_copy(dst_hbm.at[pl.ds(base + j * G, G)], didx)
                pltpu.async_copy(h_hbm.at[sidx], gbuf, sem).wait()
                pltpu.sync_copy(gbuf, acc.at[didx], add=True)
                return c

            lax.fori_loop(0, CH_AGG // G, body, 0)
            plsc.subcore_barrier()
            pltpu.sync_copy(acc.at[pl.ds(sid * WBR, WBR)],
                            out_hbm.at[pl.ds(sid * WBR, WBR)])
            plsc.subcore_barrier()


_RB = 2048
_GRID = (NP + _RB - 1) // _RB


def _shard_specs(idx_fn):
    return [pl.BlockSpec((_RB, DS), idx_fn) for _ in range(NSH)]


def _proj_tc(x, W, b):
    def body(x_ref, w_ref, b_ref, *o_refs):
        t = jnp.dot(x_ref[...], w_ref[...],
                    preferred_element_type=jnp.float32) + b_ref[...]
        for k in range(NSH):
            o_refs[k][...] = t[:, k * DS:(k + 1) * DS]

    return pl.pallas_call(
        body,
        grid=(_GRID,),
        in_specs=[
            pl.BlockSpec((_RB, D), lambda i: (i, 0)),
            pl.BlockSpec((D, D), lambda i: (0, 0)),
            pl.BlockSpec((1, D), lambda i: (0, 0)),
        ],
        out_specs=_shard_specs(lambda i: (i, 0)),
        out_shape=[jax.ShapeDtypeStruct((NP, DS), jnp.float32)
                   for _ in range(NSH)],
    )(x, W, b.reshape(1, D))


def _matmul_shards(shards, W):
    out = None
    for k in range(NSH):
        part = jnp.dot(shards[k], W[k * DS:(k + 1) * DS, :],
                       preferred_element_type=jnp.float32)
        out = part if out is None else out + part
    return out


def _recip(cnt_ref):
    c = cnt_ref[...][0, :, 0] + cnt_ref[...][1, :, 0]
    return 1.0 / jnp.maximum(c, 1.0)


def _combine_core(a_refs, ca, b_refs, cb, h_refs, wla, wlb, wrs, bsr):
    ra = _recip(ca)
    rb = _recip(cb)
    ma = [a_refs[k][...] * ra[:, None] for k in range(NSH)]
    mb = [b_refs[k][...] * rb[:, None] for k in range(NSH)]
    hh = [h_refs[k][...] for k in range(NSH)]
    t = (_matmul_shards(ma, wla[...]) + _matmul_shards(mb, wlb[...])
         + _matmul_shards(hh, wrs[...]) + bsr[...]) * 0.5
    return t


_CNT_SPEC = pl.BlockSpec((2, _RB, LANES), lambda i: (0, i, 0))
_W_SPEC = pl.BlockSpec((D, D), lambda i: (0, 0))
_B_SPEC = pl.BlockSpec((1, D), lambda i: (0, 0))


def _combine_mid_tc(agg_a, cnt_a, agg_b, cnt_b, h, Wla, Wlb, Wrs, bs):
    def body(a0, a1, a2, a3, ca, b0, b1, b2, b3, cb,
             h0, h1, h2, h3, wla, wlb, wrs, bsr, *o_refs):
        t = _combine_core((a0, a1, a2, a3), ca, (b0, b1, b2, b3), cb,
                          (h0, h1, h2, h3), wla, wlb, wrs, bsr)
        t = jnp.maximum(t, 0.0)
        for k in range(NSH):
            o_refs[k][...] = t[:, k * DS:(k + 1) * DS]

    return pl.pallas_call(
        body,
        grid=(_GRID,),
        in_specs=(_shard_specs(lambda i: (i, 0)) + [_CNT_SPEC]
                  + _shard_specs(lambda i: (i, 0)) + [_CNT_SPEC]
                  + _shard_specs(lambda i: (i, 0))
                  + [_W_SPEC, _W_SPEC, _W_SPEC, _B_SPEC]),
        out_specs=_shard_specs(lambda i: (i, 0)),
        out_shape=[jax.ShapeDtypeStruct((NP, DS), jnp.float32)
                   for _ in range(NSH)],
    )(*agg_a, cnt_a, *agg_b, cnt_b, *h, Wla, Wlb, Wrs, bs)


def _combine_cls_tc(agg_a, cnt_a, agg_b, cnt_b, h, Wla, Wlb, Wrs, bs,
                    c1w, c1b, c2w, c2b):
    def body(a0, a1, a2, a3, ca, b0, b1, b2, b3, cb,
             h0, h1, h2, h3, wla, wlb, wrs, bsr,
             w1, b1r, w2, b2r, o_ref):
        t = _combine_core((a0, a1, a2, a3), ca, (b0, b1, b2, b3), cb,
                          (h0, h1, h2, h3), wla, wlb, wrs, bsr)
        u = jnp.maximum(
            jnp.dot(t, w1[...], preferred_element_type=jnp.float32)
            + b1r[...], 0.0)
        o_ref[...] = jnp.dot(u, w2[...],
                             preferred_element_type=jnp.float32) + b2r[...]

    return pl.pallas_call(
        body,
        grid=(_GRID,),
        in_specs=(_shard_specs(lambda i: (i, 0)) + [_CNT_SPEC]
                  + _shard_specs(lambda i: (i, 0)) + [_CNT_SPEC]
                  + _shard_specs(lambda i: (i, 0))
                  + [_W_SPEC, _W_SPEC, _W_SPEC, _B_SPEC,
                     pl.BlockSpec((D, HC), lambda i: (0, 0)),
                     pl.BlockSpec((1, HC), lambda i: (0, 0)),
                     pl.BlockSpec((HC, NC_CLS), lambda i: (0, 0)),
                     pl.BlockSpec((1, NC_CLS), lambda i: (0, 0))]),
        out_specs=pl.BlockSpec((_RB, NC_CLS), lambda i: (i, 0)),
        out_shape=jax.ShapeDtypeStruct((N, NC_CLS), jnp.float32),
    )(*agg_a, cnt_a, *agg_b, cnt_b, *h, Wla, Wlb, Wrs, bs,
      c1w, c1b, c2w, c2b)


def _prep_edges(ei):
    pad = EPAD - E
    s = jnp.concatenate([ei[0], jnp.zeros((pad,), jnp.int32)])
    d = jnp.concatenate([ei[1], jnp.full((pad,), TRASH, dtype=jnp.int32)])
    return s, d


def kernel(x_transaction, x_wallet, edge_index_tt, edge_index_ww,
           edge_index_wt, edge_index_tw, tx_lin_W, tx_lin_b, wallet_lin_W,
           wallet_lin_b, Wl_tt, bl_tt, Wr_tt, Wl_ww, bl_ww, Wr_ww, Wl_wt,
           bl_wt, Wr_wt, Wl_tw, bl_tw, Wr_tw, tx_c1_W, tx_c1_b, tx_c2_W,
           tx_c2_b, w_c1_W, w_c1_b, w_c2_W, w_c2_b):
    s_tt, d_tt = _prep_edges(edge_index_tt)
    s_ww, d_ww = _prep_edges(edge_index_ww)
    s_wt, d_wt = _prep_edges(edge_index_wt)
    s_tw, d_tw = _prep_edges(edge_index_tw)

    cnt_tt = _cnt_sc(d_tt)
    cnt_ww = _cnt_sc(d_ww)
    cnt_wt = _cnt_sc(d_wt)
    cnt_tw = _cnt_sc(d_tw)

    h_tx = _proj_tc(x_transaction, tx_lin_W, tx_lin_b)
    h_w = _proj_tc(x_wallet, wallet_lin_W, wallet_lin_b)

    for l in range(2):
        agg_tt = _agg_sc(*h_tx, s_tt, d_tt)
        agg_wt = _agg_sc(*h_w, s_wt, d_wt)
        agg_ww = _agg_sc(*h_w, s_ww, d_ww)
        agg_tw = _agg_sc(*h_tx, s_tw, d_tw)
        Wrs_tx = Wr_tt[l] + Wr_wt[l]
        bs_tx = (bl_tt[l] + bl_wt[l]).reshape(1, D)
        Wrs_w = Wr_ww[l] + Wr_tw[l]
        bs_w = (bl_ww[l] + bl_tw[l]).reshape(1, D)
        if l == 0:
            h_tx = _combine_mid_tc(agg_tt, cnt_tt, agg_wt, cnt_wt, h_tx,
                                   Wl_tt[l], Wl_wt[l], Wrs_tx, bs_tx)
            h_w = _combine_mid_tc(agg_ww, cnt_ww, agg_tw, cnt_tw, h_w,
                                  Wl_ww[l], Wl_tw[l], Wrs_w, bs_w)
        else:
            out_tx = _combine_cls_tc(agg_tt, cnt_tt, agg_wt, cnt_wt, h_tx,
                                     Wl_tt[l], Wl_wt[l], Wrs_tx, bs_tx,
                                     tx_c1_W, tx_c1_b.reshape(1, HC),
                                     tx_c2_W, tx_c2_b.reshape(1, NC_CLS))
            out_w = _combine_cls_tc(agg_ww, cnt_ww, agg_tw, cnt_tw, h_w,
                                    Wl_ww[l], Wl_tw[l], Wrs_w, bs_w,
                                    w_c1_W, w_c1_b.reshape(1, HC),
                                    w_c2_W, w_c2_b.reshape(1, NC_CLS))
    return out_tx, out_w


# speedup vs baseline: 3.9198x; 2.3266x over previous
import functools

import jax
import jax.numpy as jnp
from jax import lax
from jax.experimental import pallas as pl
from jax.experimental.pallas import tpu as pltpu
from jax.experimental.pallas import tpu_sc as plsc

N = 50000
D = 128
E = 500000
HC = 64
NC_CLS = 2

NCORE = 2
NSUB = 16
LANES = 16
NSH = 4
DS = D // NSH

EPAD = 503808
CH_AGG = EPAD // NSUB
CH_CNT = EPAD // (2 * NSUB)
G = 128
NP = 50048
ACC_R = 50176
TRASH = NP
WBR = NP // NSUB
ZR = ACC_R // NSUB
ZB = 784
NCH = CH_AGG // G
IDXB = 41
NCH_CNT = CH_CNT // G

_mesh = plsc.VectorSubcoreMesh(core_axis_name="c", subcore_axis_name="s")


@functools.partial(
    pl.kernel,
    out_type=jax.ShapeDtypeStruct((2, NP, LANES), jnp.float32),
    mesh=_mesh,
    scratch_types=[
        pltpu.VMEM((NCH_CNT, G), jnp.int32),
        pltpu.VMEM((G, LANES), jnp.float32),
        pltpu.VMEM((ZB, LANES), jnp.float32),
        pltpu.VMEM_SHARED((ACC_R, LANES), jnp.float32),
    ],
    compiler_params=pltpu.CompilerParams(use_tc_tiling_on_sc=False),
)
def _cnt_sc(dst_hbm, out_hbm, didx2, obuf, zbuf, cacc):
    cid = lax.axis_index("c")
    sid = lax.axis_index("s")
    zero = jnp.zeros((LANES,), jnp.float32)
    one = jnp.ones((LANES,), jnp.float32)

    def prow(r, c):
        obuf[r, pl.ds(0, LANES)] = one
        return c

    lax.fori_loop(0, G, prow, 0)

    def zrow(r, c):
        zbuf[r, pl.ds(0, LANES)] = zero
        return c

    lax.fori_loop(0, ZB, zrow, 0)

    for k in range(ZR // ZB):
        pltpu.sync_copy(zbuf, cacc.at[pl.ds(sid * ZR + k * ZB, ZB)])
    plsc.subcore_barrier()

    wid = cid * NSUB + sid
    pltpu.sync_copy(dst_hbm.at[pl.ds(wid * NCH_CNT, NCH_CNT)], didx2)

    def body(j, c):
        pltpu.sync_copy(obuf, cacc.at[didx2.at[j]], add=True)
        return c

    lax.fori_loop(0, NCH_CNT, body, 0)
    plsc.subcore_barrier()
    pltpu.sync_copy(cacc.at[pl.ds(sid * WBR, WBR)],
                    out_hbm.at[cid, pl.ds(sid * WBR, WBR)])


@functools.partial(
    pl.kernel,
    out_type=[jax.ShapeDtypeStruct((NP, DS), jnp.float32)
              for _ in range(NSH)],
    mesh=_mesh,
    scratch_types=[
        pltpu.VMEM((IDXB, G), jnp.int32),
        pltpu.VMEM((IDXB, G), jnp.int32),
        pltpu.VMEM((G, DS), jnp.float32),
        pltpu.VMEM((G, DS), jnp.float32),
        pltpu.VMEM((G, DS), jnp.float32),
        pltpu.VMEM_SHARED((ACC_R, DS), jnp.float32),
        pltpu.SemaphoreType.DMA,
        pltpu.SemaphoreType.DMA,
    ],
    compiler_params=pltpu.CompilerParams(use_tc_tiling_on_sc=False),
)
def _agg_sc(h0, h1, h2, h3, src_hbm, dst_hbm, o0, o1, o2, o3,
            sidx2, didx2, bufa, bufb, zbuf, acc, sema, semb):
    cid = lax.axis_index("c")
    sid = lax.axis_index("s")
    zero = jnp.zeros((LANES,), jnp.float32)

    def zrow(r, c):
        for t in range(DS // LANES):
            zbuf[r, pl.ds(t * LANES, LANES)] = zero
        return c

    lax.fori_loop(0, G, zrow, 0)

    hs = (h0, h1, h2, h3)
    os = (o0, o1, o2, o3)
    for k in range(NSH):
        @pl.when(cid == k // 2)
        def _(k=k):
            h_hbm = hs[k]
            out_hbm = os[k]
            for t in range(ZR // G):
                pltpu.sync_copy(zbuf, acc.at[pl.ds(sid * ZR + t * G, G)])
            if ZR % G:
                pltpu.sync_copy(
                    zbuf.at[pl.ds(0, ZR % G)],
                    acc.at[pl.ds(sid * ZR + (ZR // G) * G, ZR % G)])
            plsc.subcore_barrier()

            def gath(j, buf, sem):
                return pltpu.async_copy(h_hbm.at[sidx2.at[j]], buf, sem)

            def gwait(buf, sem):
                pltpu.make_async_copy(h_hbm.at[sidx2.at[0]], buf, sem).wait()

            def scat(j, buf):
                pltpu.sync_copy(buf, acc.at[didx2.at[j]], add=True)

            for b in range(NCH // IDXB):
                bbase = sid * NCH + b * IDXB
                pltpu.sync_copy(src_hbm.at[pl.ds(bbase, IDXB)], sidx2)
                pltpu.sync_copy(dst_hbm.at[pl.ds(bbase, IDXB)], didx2)
                gath(0, bufa, sema)

                def body(i, c):
                    j0 = 2 * i
                    gath(j0 + 1, bufb, semb)
                    gwait(bufa, sema)
                    scat(j0, bufa)
                    gath(j0 + 2, bufa, sema)
                    gwait(bufb, semb)
                    scat(j0 + 1, bufb)
                    return c

                lax.fori_loop(0, (IDXB - 1) // 2, body, 0)
                gwait(bufa, sema)
                scat(IDXB - 1, bufa)
            plsc.subcore_barrier()
            pltpu.sync_copy(acc.at[pl.ds(sid * WBR, WBR)],
                            out_hbm.at[pl.ds(sid * WBR, WBR)])
            plsc.subcore_barrier()


_RB = 2048
_GRID = (NP + _RB - 1) // _RB


def _shard_specs(idx_fn):
    return [pl.BlockSpec((_RB, DS), idx_fn) for _ in range(NSH)]


def _proj_tc(x, W, b):
    def body(x_ref, w_ref, b_ref, *o_refs):
        t = jnp.dot(x_ref[...], w_ref[...],
                    preferred_element_type=jnp.float32) + b_ref[...]
        for k in range(NSH):
            o_refs[k][...] = t[:, k * DS:(k + 1) * DS]

    return pl.pallas_call(
        body,
        grid=(_GRID,),
        in_specs=[
            pl.BlockSpec((_RB, D), lambda i: (i, 0)),
            pl.BlockSpec((D, D), lambda i: (0, 0)),
            pl.BlockSpec((1, D), lambda i: (0, 0)),
        ],
        out_specs=_shard_specs(lambda i: (i, 0)),
        out_shape=[jax.ShapeDtypeStruct((NP, DS), jnp.float32)
                   for _ in range(NSH)],
    )(x, W, b.reshape(1, D))


def _matmul_shards(shards, W):
    out = None
    for k in range(NSH):
        part = jnp.dot(shards[k], W[k * DS:(k + 1) * DS, :],
                       preferred_element_type=jnp.float32)
        out = part if out is None else out + part
    return out


def _recip(cnt_ref):
    c = cnt_ref[...][0, :, 0] + cnt_ref[...][1, :, 0]
    return 1.0 / jnp.maximum(c, 1.0)


def _combine_core(a_refs, ca, b_refs, cb, h_refs, wla, wlb, wrs, bsr):
    ra = _recip(ca)
    rb = _recip(cb)
    ma = [a_refs[k][...] * ra[:, None] for k in range(NSH)]
    mb = [b_refs[k][...] * rb[:, None] for k in range(NSH)]
    hh = [h_refs[k][...] for k in range(NSH)]
    t = (_matmul_shards(ma, wla[...]) + _matmul_shards(mb, wlb[...])
         + _matmul_shards(hh, wrs[...]) + bsr[...]) * 0.5
    return t


_CNT_SPEC = pl.BlockSpec((2, _RB, LANES), lambda i: (0, i, 0))
_W_SPEC = pl.BlockSpec((D, D), lambda i: (0, 0))
_B_SPEC = pl.BlockSpec((1, D), lambda i: (0, 0))


def _combine_mid_tc(agg_a, cnt_a, agg_b, cnt_b, h, Wla, Wlb, Wrs, bs):
    def body(a0, a1, a2, a3, ca, b0, b1, b2, b3, cb,
             h0, h1, h2, h3, wla, wlb, wrs, bsr, *o_refs):
        t = _combine_core((a0, a1, a2, a3), ca, (b0, b1, b2, b3), cb,
                          (h0, h1, h2, h3), wla, wlb, wrs, bsr)
        t = jnp.maximum(t, 0.0)
        for k in range(NSH):
            o_refs[k][...] = t[:, k * DS:(k + 1) * DS]

    return pl.pallas_call(
        body,
        grid=(_GRID,),
        in_specs=(_shard_specs(lambda i: (i, 0)) + [_CNT_SPEC]
                  + _shard_specs(lambda i: (i, 0)) + [_CNT_SPEC]
                  + _shard_specs(lambda i: (i, 0))
                  + [_W_SPEC, _W_SPEC, _W_SPEC, _B_SPEC]),
        out_specs=_shard_specs(lambda i: (i, 0)),
        out_shape=[jax.ShapeDtypeStruct((NP, DS), jnp.float32)
                   for _ in range(NSH)],
    )(*agg_a, cnt_a, *agg_b, cnt_b, *h, Wla, Wlb, Wrs, bs)


def _combine_cls_tc(agg_a, cnt_a, agg_b, cnt_b, h, Wla, Wlb, Wrs, bs,
                    c1w, c1b, c2w, c2b):
    def body(a0, a1, a2, a3, ca, b0, b1, b2, b3, cb,
             h0, h1, h2, h3, wla, wlb, wrs, bsr,
             w1, b1r, w2, b2r, o_ref):
        t = _combine_core((a0, a1, a2, a3), ca, (b0, b1, b2, b3), cb,
                          (h0, h1, h2, h3), wla, wlb, wrs, bsr)
        u = jnp.maximum(
            jnp.dot(t, w1[...], preferred_element_type=jnp.float32)
            + b1r[...], 0.0)
        o_ref[...] = jnp.dot(u, w2[...],
                             preferred_element_type=jnp.float32) + b2r[...]

    return pl.pallas_call(
        body,
        grid=(_GRID,),
        in_specs=(_shard_specs(lambda i: (i, 0)) + [_CNT_SPEC]
                  + _shard_specs(lambda i: (i, 0)) + [_CNT_SPEC]
                  + _shard_specs(lambda i: (i, 0))
                  + [_W_SPEC, _W_SPEC, _W_SPEC, _B_SPEC,
                     pl.BlockSpec((D, HC), lambda i: (0, 0)),
                     pl.BlockSpec((1, HC), lambda i: (0, 0)),
                     pl.BlockSpec((HC, NC_CLS), lambda i: (0, 0)),
                     pl.BlockSpec((1, NC_CLS), lambda i: (0, 0))]),
        out_specs=pl.BlockSpec((_RB, NC_CLS), lambda i: (i, 0)),
        out_shape=jax.ShapeDtypeStruct((N, NC_CLS), jnp.float32),
    )(*agg_a, cnt_a, *agg_b, cnt_b, *h, Wla, Wlb, Wrs, bs,
      c1w, c1b, c2w, c2b)


def _prep_edges(ei):
    pad = EPAD - E
    s = jnp.concatenate([ei[0], jnp.zeros((pad,), jnp.int32)])
    d = jnp.concatenate([ei[1], jnp.full((pad,), TRASH, dtype=jnp.int32)])
    return s.reshape(EPAD // G, G), d.reshape(EPAD // G, G)


def kernel(x_transaction, x_wallet, edge_index_tt, edge_index_ww,
           edge_index_wt, edge_index_tw, tx_lin_W, tx_lin_b, wallet_lin_W,
           wallet_lin_b, Wl_tt, bl_tt, Wr_tt, Wl_ww, bl_ww, Wr_ww, Wl_wt,
           bl_wt, Wr_wt, Wl_tw, bl_tw, Wr_tw, tx_c1_W, tx_c1_b, tx_c2_W,
           tx_c2_b, w_c1_W, w_c1_b, w_c2_W, w_c2_b):
    s_tt, d_tt = _prep_edges(edge_index_tt)
    s_ww, d_ww = _prep_edges(edge_index_ww)
    s_wt, d_wt = _prep_edges(edge_index_wt)
    s_tw, d_tw = _prep_edges(edge_index_tw)

    cnt_tt = _cnt_sc(d_tt)
    cnt_ww = _cnt_sc(d_ww)
    cnt_wt = _cnt_sc(d_wt)
    cnt_tw = _cnt_sc(d_tw)

    h_tx = _proj_tc(x_transaction, tx_lin_W, tx_lin_b)
    h_w = _proj_tc(x_wallet, wallet_lin_W, wallet_lin_b)

    for l in range(2):
        agg_tt = _agg_sc(*h_tx, s_tt, d_tt)
        agg_wt = _agg_sc(*h_w, s_wt, d_wt)
        agg_ww = _agg_sc(*h_w, s_ww, d_ww)
        agg_tw = _agg_sc(*h_tx, s_tw, d_tw)
        Wrs_tx = Wr_tt[l] + Wr_wt[l]
        bs_tx = (bl_tt[l] + bl_wt[l]).reshape(1, D)
        Wrs_w = Wr_ww[l] + Wr_tw[l]
        bs_w = (bl_ww[l] + bl_tw[l]).reshape(1, D)
        if l == 0:
            h_tx = _combine_mid_tc(agg_tt, cnt_tt, agg_wt, cnt_wt, h_tx,
                                   Wl_tt[l], Wl_wt[l], Wrs_tx, bs_tx)
            h_w = _combine_mid_tc(agg_ww, cnt_ww, agg_tw, cnt_tw, h_w,
                                  Wl_ww[l], Wl_tw[l], Wrs_w, bs_w)
        else:
            out_tx = _combine_cls_tc(agg_tt, cnt_tt, agg_wt, cnt_wt, h_tx,
                                     Wl_tt[l], Wl_wt[l], Wrs_tx, bs_tx,
                                     tx_c1_W, tx_c1_b.reshape(1, HC),
                                     tx_c2_W, tx_c2_b.reshape(1, NC_CLS))
            out_w = _combine_cls_tc(agg_ww, cnt_ww, agg_tw, cnt_tw, h_w,
                                    Wl_ww[l], Wl_tw[l], Wrs_w, bs_w,
                                    w_c1_W, w_c1_b.reshape(1, HC),
                                    w_c2_W, w_c2_b.reshape(1, NC_CLS))
    return out_tx, out_w
